# baseline (speedup 1.0000x reference)
import jax
import jax.numpy as jnp
from jax.experimental import pallas as pl
from jax.experimental.pallas import tpu as pltpu

N, EH, D, H = 10000, 2000, 256, 256
TN = 1000
NT = N // TN
_CDT = jnp.bfloat16


def _mlp(t, W1, b1, W2, b2, Wc):
    h = jnp.dot(t, W1[...], preferred_element_type=jnp.float32) + b1[...]
    h = jnp.maximum(h, 0.0).astype(_CDT)
    h = jnp.dot(h, W2[...], preferred_element_type=jnp.float32) + b2[...]
    h = jnp.maximum(h, 0.0).astype(_CDT)
    return jnp.dot(h, Wc[...], preferred_element_type=jnp.float32)


def _body(x0_ref, inc_ref,
          v0W1, v0b1, v0W2, v0b2, v0Wc,
          e0W1, e0b1, e0W2, e0b2, e0Wc,
          v1W1, v1b1, v1W2, v1b2, v1Wc,
          e1W1, e1b1, e1W2, e1b2, e1Wc,
          linW, linb,
          out_ref, xe0, hce0, xe1, hce1, mx):
    p = pl.program_id(0)
    i = pl.program_id(1)
    inc = inc_ref[...]

    @pl.when(p == 0)
    def _pass0():
        @pl.when(i == 0)
        def _():
            xe0[...] = jnp.zeros_like(xe0)

        hc = _mlp(x0_ref[...], v0W1, v0b1, v0W2, v0b2, v0Wc).astype(_CDT)
        xe0[...] += jax.lax.dot_general(
            inc, hc, (((0,), (0,)), ((), ())),
            preferred_element_type=jnp.float32)

    @pl.when(p == 1)
    def _pass1():
        @pl.when(i == 0)
        def _():
            xe = jnp.maximum(xe0[...], 0.0).astype(_CDT)
            hce0[...] = _mlp(xe, e0W1, e0b1, e0W2, e0b2, e0Wc).astype(_CDT)
            xe1[...] = jnp.zeros_like(xe1)

        x1 = jnp.maximum(
            jnp.dot(inc, hce0[...], preferred_element_type=jnp.float32), 0.0)
        hc1 = _mlp(x1.astype(_CDT), v1W1, v1b1, v1W2, v1b2, v1Wc).astype(_CDT)
        xe1[...] += jax.lax.dot_general(
            inc, hc1, (((0,), (0,)), ((), ())),
            preferred_element_type=jnp.float32)

    @pl.when(p == 2)
    def _pass2():
        @pl.when(i == 0)
        def _():
            xe = jnp.maximum(xe1[...], 0.0).astype(_CDT)
            hce1[...] = _mlp(xe, e1W1, e1b1, e1W2, e1b2, e1Wc).astype(_CDT)
            mx[...] = jnp.full_like(mx, -jnp.inf)

        x2 = jnp.maximum(
            jnp.dot(inc, hce1[...], preferred_element_type=jnp.float32), 0.0)
        mx[...] = jnp.maximum(mx[...], jnp.max(x2, axis=0, keepdims=True))

        @pl.when(i == NT - 1)
        def _():
            out_ref[...] = (jnp.dot(mx[...], linW[...],
                                    preferred_element_type=jnp.float32)
                            + linb[...])


def kernel(x_0, incidence_1, l0_v2e_W1, l0_v2e_b1, l0_v2e_W2, l0_v2e_b2,
           l0_v2e_Wc, l0_e2v_W1, l0_e2v_b1, l0_e2v_W2, l0_e2v_b2, l0_e2v_Wc,
           l1_v2e_W1, l1_v2e_b1, l1_v2e_W2, l1_v2e_b2, l1_v2e_Wc,
           l1_e2v_W1, l1_e2v_b1, l1_e2v_W2, l1_e2v_b2, l1_e2v_Wc,
           lin_W, lin_b):
    x0 = x_0.astype(_CDT)
    inc = incidence_1.astype(_CDT)

    def w(a):
        return a.astype(_CDT)

    def b(a):
        return a.reshape(1, H).astype(jnp.float32)

    weights = [
        w(l0_v2e_W1), b(l0_v2e_b1), w(l0_v2e_W2), b(l0_v2e_b2), w(l0_v2e_Wc),
        w(l0_e2v_W1), b(l0_e2v_b1), w(l0_e2v_W2), b(l0_e2v_b2), w(l0_e2v_Wc),
        w(l1_v2e_W1), b(l1_v2e_b1), w(l1_v2e_W2), b(l1_v2e_b2), w(l1_v2e_Wc),
        w(l1_e2v_W1), b(l1_e2v_b1), w(l1_e2v_W2), b(l1_e2v_b2), w(l1_e2v_Wc),
        lin_W.astype(jnp.float32), lin_b.reshape(1, H).astype(jnp.float32),
    ]

    full = lambda shape: pl.BlockSpec(shape, lambda p, i: (0, 0))
    in_specs = [
        pl.BlockSpec((TN, D), lambda p, i: (jnp.where(p == 0, i, 0), 0)),
        pl.BlockSpec((TN, EH), lambda p, i: (i, 0)),
    ] + [full(a.shape) for a in weights]

    out = pl.pallas_call(
        _body,
        grid=(3, NT),
        in_specs=in_specs,
        out_specs=pl.BlockSpec((1, H), lambda p, i: (0, 0)),
        out_shape=jax.ShapeDtypeStruct((1, H), jnp.float32),
        scratch_shapes=[
            pltpu.VMEM((EH, H), jnp.float32),
            pltpu.VMEM((EH, H), _CDT),
            pltpu.VMEM((EH, H), jnp.float32),
            pltpu.VMEM((EH, H), _CDT),
            pltpu.VMEM((1, H), jnp.float32),
        ],
        compiler_params=pltpu.CompilerParams(
            dimension_semantics=("arbitrary", "arbitrary"),
            vmem_limit_bytes=100 * 1024 * 1024,
        ),
    )(x0, inc, *weights)
    return out.reshape(H)


# speedup vs baseline: 1.0401x; 1.0401x over previous
import jax
import jax.numpy as jnp
from jax.experimental import pallas as pl
from jax.experimental.pallas import tpu as pltpu

N, EH, D, H = 10000, 2000, 256, 256
TN = 1000
NT = N // TN
_CDT = jnp.bfloat16

_C00 = (((0,), (0,)), ((), ()))
_C11 = (((1,), (1,)), ((), ()))


def _dot(a, b, dims):
    return jax.lax.dot_general(a, b, dims, preferred_element_type=jnp.float32)


def _mlp(t, W1, b1, W2, b2, Wc):
    h = jnp.dot(t, W1[...], preferred_element_type=jnp.float32) + b1[...]
    h = jnp.maximum(h, 0.0).astype(_CDT)
    h = jnp.dot(h, W2[...], preferred_element_type=jnp.float32) + b2[...]
    h = jnp.maximum(h, 0.0).astype(_CDT)
    return jnp.dot(h, Wc[...], preferred_element_type=jnp.float32)


def _mlp_t(tT, W1, b1, W2, b2, Wc):
    h = _dot(W1[...], tT, _C00) + b1[...]
    h = jnp.maximum(h, 0.0).astype(_CDT)
    h = _dot(W2[...], h, _C00) + b2[...]
    h = jnp.maximum(h, 0.0).astype(_CDT)
    return _dot(Wc[...], h, _C00)


def _body(x0_ref, inc_ref,
          v0W1, v0b1, v0W2, v0b2, v0Wc,
          e0W1, e0b1, e0W2, e0b2, e0Wc,
          v1W1, v1b1, v1W2, v1b2, v1Wc,
          e1W1, e1b1, e1W2, e1b2, e1Wc,
          linW, linb,
          out_ref, xe0T, hce0T, xe1T, hce1T, mxT):
    p = pl.program_id(0)
    i = pl.program_id(1)
    inc = inc_ref[...]

    @pl.when(p == 0)
    def _pass0():
        @pl.when(i == 0)
        def _():
            xe0T[...] = jnp.zeros_like(xe0T)

        hc = _mlp(x0_ref[...], v0W1, v0b1, v0W2, v0b2, v0Wc).astype(_CDT)
        xe0T[...] += _dot(hc, inc, _C00)

    @pl.when(p == 1)
    def _pass1():
        @pl.when(i == 0)
        def _():
            xeT = jnp.maximum(xe0T[...], 0.0).astype(_CDT)
            hce0T[...] = _mlp_t(xeT, e0W1, e0b1, e0W2, e0b2,
                                e0Wc).astype(_CDT)
            xe1T[...] = jnp.zeros_like(xe1T)

        x1T = jnp.maximum(_dot(hce0T[...], inc, _C11), 0.0)
        hc1T = _mlp_t(x1T.astype(_CDT), v1W1, v1b1, v1W2, v1b2,
                      v1Wc).astype(_CDT)
        xe1T[...] += jnp.dot(hc1T, inc, preferred_element_type=jnp.float32)

    @pl.when(p == 2)
    def _pass2():
        @pl.when(i == 0)
        def _():
            xeT = jnp.maximum(xe1T[...], 0.0).astype(_CDT)
            hce1T[...] = _mlp_t(xeT, e1W1, e1b1, e1W2, e1b2,
                                e1Wc).astype(_CDT)
            mxT[...] = jnp.full_like(mxT, -jnp.inf)

        x2T = jnp.maximum(_dot(hce1T[...], inc, _C11), 0.0)
        mxT[...] = jnp.maximum(mxT[...], x2T)

        @pl.when(i == NT - 1)
        def _():
            mx = jnp.max(mxT[...], axis=1, keepdims=True)
            out_ref[...] = _dot(mx.astype(jnp.float32), linW[...],
                                _C00) + linb[...]


def kernel(x_0, incidence_1, l0_v2e_W1, l0_v2e_b1, l0_v2e_W2, l0_v2e_b2,
           l0_v2e_Wc, l0_e2v_W1, l0_e2v_b1, l0_e2v_W2, l0_e2v_b2, l0_e2v_Wc,
           l1_v2e_W1, l1_v2e_b1, l1_v2e_W2, l1_v2e_b2, l1_v2e_Wc,
           l1_e2v_W1, l1_e2v_b1, l1_e2v_W2, l1_e2v_b2, l1_e2v_Wc,
           lin_W, lin_b):
    x0 = x_0.astype(_CDT)
    inc = incidence_1.astype(_CDT)

    def w(a):
        return a.astype(_CDT)

    def brow(a):
        return a.reshape(1, H).astype(jnp.float32)

    def bcol(a):
        return a.reshape(H, 1).astype(jnp.float32)

    weights = [
        w(l0_v2e_W1), brow(l0_v2e_b1), w(l0_v2e_W2), brow(l0_v2e_b2),
        w(l0_v2e_Wc),
        w(l0_e2v_W1), bcol(l0_e2v_b1), w(l0_e2v_W2), bcol(l0_e2v_b2),
        w(l0_e2v_Wc),
        w(l1_v2e_W1), bcol(l1_v2e_b1), w(l1_v2e_W2), bcol(l1_v2e_b2),
        w(l1_v2e_Wc),
        w(l1_e2v_W1), bcol(l1_e2v_b1), w(l1_e2v_W2), bcol(l1_e2v_b2),
        w(l1_e2v_Wc),
        lin_W.astype(jnp.float32), lin_b.reshape(1, H).astype(jnp.float32),
    ]

    full = lambda shape: pl.BlockSpec(shape, lambda p, i: (0, 0))
    in_specs = [
        pl.BlockSpec((TN, D), lambda p, i: (jnp.where(p == 0, i, 0), 0)),
        pl.BlockSpec((TN, EH), lambda p, i: (i, 0)),
    ] + [full(a.shape) for a in weights]

    out = pl.pallas_call(
        _body,
        grid=(3, NT),
        in_specs=in_specs,
        out_specs=pl.BlockSpec((1, H), lambda p, i: (0, 0)),
        out_shape=jax.ShapeDtypeStruct((1, H), jnp.float32),
        scratch_shapes=[
            pltpu.VMEM((H, EH), jnp.float32),
            pltpu.VMEM((H, EH), _CDT),
            pltpu.VMEM((H, EH), jnp.float32),
            pltpu.VMEM((H, EH), _CDT),
            pltpu.VMEM((H, TN), jnp.float32),
        ],
        compiler_params=pltpu.CompilerParams(
            dimension_semantics=("arbitrary", "arbitrary"),
            vmem_limit_bytes=100 * 1024 * 1024,
        ),
    )(x0, inc, *weights)
    return out.reshape(H)
